# baseline (speedup 1.0000x reference)
import functools

import jax
import jax.numpy as jnp
from jax import lax
from jax.experimental import pallas as pl
from jax.experimental.pallas import tpu as pltpu
from jax.experimental.pallas import tpu_sc as plsc

_M = 1000000
_D = 64
_B = 16384
_LAMB = 10.0
_L = 16
_NW = 32
_TILE = 128
_NFULL = _M // _TILE
_TPW = _NFULL // _NW
_W = 256
_CW = 288
_IDBLK = 2048

_mesh = plsc.VectorSubcoreMesh(core_axis_name="c", subcore_axis_name="s")

_SPLAT_DNUMS = lax.GatherDimensionNumbers(
    offset_dims=(), collapsed_slice_dims=(0,), start_index_map=(0,))


def _dsplat(ref, j):
    base = (j // _L) * _L
    v = ref[pl.ds(base, _L)]
    idx = jnp.full((_L, 1), j - base, jnp.int32)
    return lax.gather(v, idx, _SPLAT_DNUMS, (1,),
                      mode=lax.GatherScatterMode.PROMISE_IN_BOUNDS)


def _dscalar(ref, j):
    return lax.squeeze(lax.slice(_dsplat(ref, j), (0,), (1,)), (0,))


@functools.partial(
    pl.kernel,
    out_type=(jax.ShapeDtypeStruct((_D, _M), jnp.float32),
              jax.ShapeDtypeStruct((_M,), jnp.float32)),
    mesh=_mesh,
    compiler_params=pltpu.CompilerParams(use_tc_tiling_on_sc=True,
                                         needs_layout_passes=False),
    scratch_types=[
        pltpu.VMEM((_IDBLK,), jnp.int32),
        pltpu.VMEM((_B,), jnp.int32),
        pltpu.VMEM((_B,), jnp.int32),
        pltpu.VMEM((_B,), jnp.float32),
        pltpu.VMEM((128,), jnp.float32),
        pltpu.VMEM((128,), jnp.float32),
        pltpu.VMEM((_D, _W), jnp.float32),
        pltpu.VMEM((_W,), jnp.float32),
        pltpu.VMEM((_CW,), jnp.int32),
        pltpu.VMEM((_CW,), jnp.int32),
        pltpu.VMEM((_CW,), jnp.float32),
        pltpu.VMEM((_CW,), jnp.float32),
        pltpu.VMEM((_W, _D), jnp.float32),
        pltpu.SemaphoreType.DMA,
        pltpu.SemaphoreType.DMA,
        pltpu.SemaphoreType.DMA,
    ],
)
def _sc_update(memT, lu_in, ids_hbm, msg_hbm, ts_hbm, outT, lu_out,
               idsbuf, mid, mbidx, mf, luv, tsv, colbuf, lubuf,
               cmid, cbidx, cf, cts, msgbuf, sem_a, sem_b, sem_c):
    wid = lax.axis_index("s") * 2 + lax.axis_index("c")
    ntiles = _TPW + jnp.where(wid < 4, 1, 0)
    lo = _TILE * (_TPW * wid + jnp.minimum(wid, 4))
    ncols = _TILE * ntiles + jnp.where(wid == _NW - 1, _M - _TILE * _NFULL, 0)
    hi = lo + ncols
    zeros = jnp.zeros((_L,), jnp.int32)

    def scan_blk(blk, cnt):
        pltpu.sync_copy(ids_hbm.at[pl.ds(blk * _IDBLK, _IDBLK)], idsbuf)

        def scan_v(g, cnt):
            v = idsbuf[pl.ds(g * _L, _L)]
            pos = blk * _IDBLK + g * _L + lax.iota(jnp.int32, _L)
            m = (v >= lo) & (v < hi)
            plsc.store_compressed(mid.at[pl.ds(cnt, _L)], v, mask=m)
            plsc.store_compressed(mbidx.at[pl.ds(cnt, _L)], pos, mask=m)
            return cnt + plsc.all_reduce_population_count(m)[0]

        return lax.fori_loop(0, _IDBLK // _L, scan_v, cnt)

    cnt = lax.fori_loop(0, _B // _IDBLK, scan_blk, 0)

    def zpad(g, carry):
        p = cnt + g * _L
        mid[pl.ds(p, _L)] = zeros
        mbidx[pl.ds(p, _L)] = zeros
        return carry

    lax.fori_loop(0, 128 // _L + 1, zpad, 0)

    def fblk(b, carry):
        p = b * 128
        pltpu.async_copy(lu_in.at[mid.at[pl.ds(p, 128)]], luv, sem_a).wait()
        pltpu.async_copy(ts_hbm.at[mbidx.at[pl.ds(p, 128)]], tsv, sem_a).wait()
        for q in range(128 // _L):
            s = pl.ds(q * _L, _L)
            mf[pl.ds(p + q * _L, _L)] = jnp.exp(
                (luv[s] - tsv[s]) * (1.0 / _LAMB))
        return carry

    lax.fori_loop(0, (cnt + 127) // 128, fblk, 0)

    def chunk(c0, w):
        pltpu.async_copy(memT.at[:, pl.ds(c0, w)],
                         colbuf.at[:, pl.ds(0, w)], sem_a)
        pltpu.async_copy(lu_in.at[pl.ds(c0, w)],
                         lubuf.at[pl.ds(0, w)], sem_b)
        for g in range(_CW // _L):
            cbidx[pl.ds(g * _L, _L)] = zeros

        def cscan(g, ccnt):
            base = g * _L
            v = mid[pl.ds(base, _L)]
            valid = (base + lax.iota(jnp.int32, _L)) < cnt
            m = (v >= c0) & (v < c0 + w) & valid
            plsc.store_compressed(cmid.at[pl.ds(ccnt, _L)], v, mask=m)
            plsc.store_compressed(cbidx.at[pl.ds(ccnt, _L)],
                                  mbidx[pl.ds(base, _L)], mask=m)
            plsc.store_compressed(cf.at[pl.ds(ccnt, _L)],
                                  mf[pl.ds(base, _L)], mask=m)
            return ccnt + plsc.all_reduce_population_count(m)[0]

        ccnt = lax.fori_loop(0, (cnt + _L - 1) // _L, cscan, 0)

        def tsg(b, carry):
            pltpu.async_copy(ts_hbm.at[cbidx.at[pl.ds(b * 128, 128)]],
                             cts.at[pl.ds(b * 128, 128)], sem_c)
            return carry

        ntb = (ccnt + 127) // 128
        lax.fori_loop(0, ntb, tsg, 0)

        def msg_fire(j, carry):
            b = _dscalar(cbidx, j)
            pltpu.async_copy(msg_hbm.at[pl.ds(b * _D, _D)],
                             msgbuf.at[j], sem_b)
            return carry

        lax.fori_loop(0, ccnt, msg_fire, 0)

        def tsg_drain(b, carry):
            pltpu.make_async_copy(ts_hbm.at[pl.ds(0, 128)],
                                  cts.at[pl.ds(0, 128)], sem_c).wait()
            return carry

        lax.fori_loop(0, ntb, tsg_drain, 0)

        def msg_drain(j, carry):
            pltpu.make_async_copy(msg_hbm.at[pl.ds(0, _D)],
                                  msgbuf.at[0], sem_b).wait()
            return carry

        pltpu.make_async_copy(lu_in.at[pl.ds(c0, w)],
                              lubuf.at[pl.ds(0, w)], sem_b).wait()
        lax.fori_loop(0, ccnt, msg_drain, 0)
        pltpu.make_async_copy(memT.at[:, pl.ds(c0, w)],
                              colbuf.at[:, pl.ds(0, w)], sem_a).wait()

        def apply(j, carry):
            cl = _dsplat(cmid, j) - c0
            fs = _dsplat(cf, j)
            for q in range(_D // _L):
                rows = lax.iota(jnp.int32, _L) + q * _L
                old = plsc.load_gather(colbuf, [rows, cl])
                msgv = msgbuf[j, pl.ds(q * _L, _L)]
                plsc.store_scatter(colbuf, [rows, cl], msgv + fs * old)
            return carry

        lax.fori_loop(0, ccnt, apply, 0)

        def luapp(g, carry):
            base = g * _L
            v = cmid[pl.ds(base, _L)] - c0
            valid = (base + lax.iota(jnp.int32, _L)) < ccnt
            plsc.store_scatter(lubuf, [v], cts[pl.ds(base, _L)], mask=valid)
            return carry

        lax.fori_loop(0, (ccnt + _L - 1) // _L, luapp, 0)

        pltpu.async_copy(colbuf.at[:, pl.ds(0, w)],
                         outT.at[:, pl.ds(c0, w)], sem_a).wait()
        pltpu.async_copy(lubuf.at[pl.ds(0, w)],
                         lu_out.at[pl.ds(c0, w)], sem_b).wait()

    def chunk_loop(ch, carry):
        chunk(lo + ch * _W, _W)
        return carry

    nch = (_TILE * _TPW) // _W
    lax.fori_loop(0, nch, chunk_loop, 0)

    @pl.when(wid < 4)
    def _rem():
        chunk(lo + nch * _W, _TILE)

    @pl.when(wid == _NW - 1)
    def _lu_tail():
        c0 = lo + nch * _W
        w = _M - _TILE * _NFULL
        pltpu.async_copy(lu_in.at[pl.ds(c0, w)],
                         lubuf.at[pl.ds(0, w)], sem_b).wait()

        def cscan(g, ccnt):
            base = g * _L
            v = mid[pl.ds(base, _L)]
            valid = (base + lax.iota(jnp.int32, _L)) < cnt
            m = (v >= c0) & valid
            plsc.store_compressed(cmid.at[pl.ds(ccnt, _L)], v, mask=m)
            plsc.store_compressed(cbidx.at[pl.ds(ccnt, _L)],
                                  mbidx[pl.ds(base, _L)], mask=m)
            return ccnt + plsc.all_reduce_population_count(m)[0]

        for g in range(_CW // _L):
            cbidx[pl.ds(g * _L, _L)] = zeros
        ccnt = lax.fori_loop(0, (cnt + _L - 1) // _L, cscan, 0)
        pltpu.async_copy(ts_hbm.at[cbidx.at[pl.ds(0, 128)]],
                         cts.at[pl.ds(0, 128)], sem_c).wait()

        def luapp(g, carry):
            base = g * _L
            v = cmid[pl.ds(base, _L)] - c0
            valid = (base + lax.iota(jnp.int32, _L)) < ccnt
            plsc.store_scatter(lubuf, [v], cts[pl.ds(base, _L)], mask=valid)
            return carry

        lax.fori_loop(0, (ccnt + _L - 1) // _L, luapp, 0)
        pltpu.async_copy(lubuf.at[pl.ds(0, w)],
                         lu_out.at[pl.ds(c0, w)], sem_b).wait()


_TC0 = _TILE * _NFULL
_TW = _M - _TC0
_TWB = 128
_KB = 1024
_KG = _B // _KB


def _tc_tail_body(memT_ref, msgT_ref, ids_ref, ts_ref, lut_ref, alias_ref,
                  out_ref, acc_msg, acc_ts, acc_hit):
    k = pl.program_id(0)

    @pl.when(k == 0)
    def _init():
        acc_msg[...] = jnp.zeros((_D, _TWB), jnp.float32)
        acc_ts[...] = jnp.zeros((1, _TWB), jnp.float32)
        acc_hit[...] = jnp.zeros((1, _TWB), jnp.float32)

    ids_b = ids_ref[0]
    ts_b = ts_ref[0]
    tgt = _TC0 + lax.broadcasted_iota(jnp.int32, (_TWB, 1), 0)
    oh = (ids_b == tgt).astype(jnp.float32)
    dn = (((1,), (1,)), ((), ()))
    acc_msg[...] += lax.dot_general(msgT_ref[...], oh, dn,
                                    preferred_element_type=jnp.float32)
    acc_ts[...] += lax.dot_general(ts_b, oh, dn,
                                   preferred_element_type=jnp.float32)
    ones = jnp.ones((1, _KB), jnp.float32)
    acc_hit[...] += lax.dot_general(ones, oh, dn,
                                    preferred_element_type=jnp.float32)

    @pl.when(k == _KG - 1)
    def _finish():
        old = memT_ref[...]
        lut = lut_ref[...]
        f = jnp.exp((lut - acc_ts[...]) * (1.0 / _LAMB))
        new = acc_msg[...] + f * old
        out_ref[...] = jnp.where(acc_hit[...] > 0.0, new, old)


_tc_tail = pl.pallas_call(
    _tc_tail_body,
    grid=(_KG,),
    in_specs=[
        pl.BlockSpec((_D, _TWB), lambda k: (0, _TC0 // _TWB)),
        pl.BlockSpec((_D, _KB), lambda k: (0, k)),
        pl.BlockSpec((1, 1, _KB), lambda k: (k, 0, 0)),
        pl.BlockSpec((1, 1, _KB), lambda k: (k, 0, 0)),
        pl.BlockSpec((1, _TWB), lambda k: (0, 0)),
        pl.BlockSpec((_D, _TWB), lambda k: (0, _TC0 // _TWB)),
    ],
    out_specs=pl.BlockSpec((_D, _TWB), lambda k: (0, _TC0 // _TWB)),
    out_shape=jax.ShapeDtypeStruct((_D, _M), jnp.float32),
    scratch_shapes=[
        pltpu.VMEM((_D, _TWB), jnp.float32),
        pltpu.VMEM((1, _TWB), jnp.float32),
        pltpu.VMEM((1, _TWB), jnp.float32),
    ],
    input_output_aliases={5: 0},
)


def kernel(memory, last_update, unique_node_ids, unique_messages, timestamps):
    memT = memory.T
    msg_flat = unique_messages.reshape(-1)
    outT, lu_out = _sc_update(memT, last_update, unique_node_ids,
                              msg_flat, timestamps)
    ids3 = unique_node_ids.reshape(_KG, 1, _KB)
    ts3 = timestamps.reshape(_KG, 1, _KB)
    lut = jnp.pad(lax.dynamic_slice(last_update, (_TC0,), (_TW,)),
                  (0, _TWB - _TW)).reshape(1, _TWB)
    outT = _tc_tail(memT, unique_messages.T, ids3, ts3, lut, outT)
    return outT.T, lu_out


# speedup vs baseline: 1.0079x; 1.0079x over previous
import functools

import jax
import jax.numpy as jnp
from jax import lax
from jax.experimental import pallas as pl
from jax.experimental.pallas import tpu as pltpu
from jax.experimental.pallas import tpu_sc as plsc

_M = 1000000
_D = 64
_B = 16384
_LAMB = 10.0
_L = 16
_NW = 32
_TILE = 128
_NFULL = _M // _TILE
_TPW = _NFULL // _NW
_W = 256
_CW = 288
_IDBLK = 2048

_mesh = plsc.VectorSubcoreMesh(core_axis_name="c", subcore_axis_name="s")

_SPLAT_DNUMS = lax.GatherDimensionNumbers(
    offset_dims=(), collapsed_slice_dims=(0,), start_index_map=(0,))


def _dsplat(ref, j):
    base = (j // _L) * _L
    v = ref[pl.ds(base, _L)]
    idx = jnp.full((_L, 1), j - base, jnp.int32)
    return lax.gather(v, idx, _SPLAT_DNUMS, (1,),
                      mode=lax.GatherScatterMode.PROMISE_IN_BOUNDS)


def _dscalar(ref, j):
    return lax.squeeze(lax.slice(_dsplat(ref, j), (0,), (1,)), (0,))


@functools.partial(
    pl.kernel,
    out_type=(jax.ShapeDtypeStruct((_D, _M), jnp.float32),
              jax.ShapeDtypeStruct((_M,), jnp.float32)),
    mesh=_mesh,
    compiler_params=pltpu.CompilerParams(use_tc_tiling_on_sc=True,
                                         needs_layout_passes=False),
    scratch_types=[
        pltpu.VMEM((_IDBLK,), jnp.int32),
        pltpu.VMEM((_B,), jnp.int32),
        pltpu.VMEM((_B,), jnp.int32),
        pltpu.VMEM((_B,), jnp.float32),
        pltpu.VMEM((128,), jnp.float32),
        pltpu.VMEM((128,), jnp.float32),
        pltpu.VMEM((_D, _W), jnp.float32),
        pltpu.VMEM((_D, _W), jnp.float32),
        pltpu.VMEM((_W,), jnp.float32),
        pltpu.VMEM((_W,), jnp.float32),
        pltpu.VMEM((_CW,), jnp.int32),
        pltpu.VMEM((_CW,), jnp.int32),
        pltpu.VMEM((_CW,), jnp.float32),
        pltpu.VMEM((_CW,), jnp.float32),
        pltpu.VMEM((_W, _D), jnp.float32),
        pltpu.SemaphoreType.DMA,
        pltpu.SemaphoreType.DMA,
        pltpu.SemaphoreType.DMA,
        pltpu.SemaphoreType.DMA,
        pltpu.SemaphoreType.DMA,
        pltpu.SemaphoreType.DMA,
    ],
)
def _sc_update(memT, lu_in, ids_hbm, msg_hbm, ts_hbm, outT, lu_out,
               idsbuf, mid, mbidx, mf, luv, tsv, colbuf0, colbuf1, lubuf0, lubuf1,
               cmid, cbidx, cf, cts, msgbuf, si0, si1, so0, so1, sm, st):
    wid = lax.axis_index("s") * 2 + lax.axis_index("c")
    ntiles = _TPW + jnp.where(wid < 4, 1, 0)
    lo = _TILE * (_TPW * wid + jnp.minimum(wid, 4))
    ncols = _TILE * ntiles + jnp.where(wid == _NW - 1, _M - _TILE * _NFULL, 0)
    hi = lo + ncols
    zeros = jnp.zeros((_L,), jnp.int32)

    def scan_blk(blk, cnt):
        pltpu.sync_copy(ids_hbm.at[pl.ds(blk * _IDBLK, _IDBLK)], idsbuf)

        def scan_v(g, cnt):
            v = idsbuf[pl.ds(g * _L, _L)]
            pos = blk * _IDBLK + g * _L + lax.iota(jnp.int32, _L)
            m = (v >= lo) & (v < hi)
            plsc.store_compressed(mid.at[pl.ds(cnt, _L)], v, mask=m)
            plsc.store_compressed(mbidx.at[pl.ds(cnt, _L)], pos, mask=m)
            return cnt + plsc.all_reduce_population_count(m)[0]

        return lax.fori_loop(0, _IDBLK // _L, scan_v, cnt)

    cnt = lax.fori_loop(0, _B // _IDBLK, scan_blk, 0)

    def zpad(g, carry):
        p = cnt + g * _L
        mid[pl.ds(p, _L)] = zeros
        mbidx[pl.ds(p, _L)] = zeros
        return carry

    lax.fori_loop(0, 128 // _L + 1, zpad, 0)

    def fblk(b, carry):
        p = b * 128
        pltpu.async_copy(lu_in.at[mid.at[pl.ds(p, 128)]], luv, si0).wait()
        pltpu.async_copy(ts_hbm.at[mbidx.at[pl.ds(p, 128)]], tsv, si0).wait()
        for q in range(128 // _L):
            s = pl.ds(q * _L, _L)
            mf[pl.ds(p + q * _L, _L)] = jnp.exp(
                (luv[s] - tsv[s]) * (1.0 / _LAMB))
        return carry

    lax.fori_loop(0, (cnt + 127) // 128, fblk, 0)

    def fire_in(c0, cb, lb, sem, w):
        pltpu.async_copy(memT.at[:, pl.ds(c0, w)], cb.at[:, pl.ds(0, w)], sem)
        pltpu.async_copy(lu_in.at[pl.ds(c0, w)], lb.at[pl.ds(0, w)], sem)

    def wait_in(c0, cb, lb, sem, w):
        pltpu.make_async_copy(memT.at[:, pl.ds(c0, w)],
                              cb.at[:, pl.ds(0, w)], sem).wait()
        pltpu.make_async_copy(lu_in.at[pl.ds(c0, w)],
                              lb.at[pl.ds(0, w)], sem).wait()

    def fire_out(c0, cb, lb, sem, w):
        pltpu.async_copy(cb.at[:, pl.ds(0, w)], outT.at[:, pl.ds(c0, w)], sem)
        pltpu.async_copy(lb.at[pl.ds(0, w)], lu_out.at[pl.ds(c0, w)], sem)

    def wait_out(c0, cb, lb, sem, w):
        pltpu.make_async_copy(cb.at[:, pl.ds(0, w)],
                              outT.at[:, pl.ds(c0, w)], sem).wait()
        pltpu.make_async_copy(lb.at[pl.ds(0, w)],
                              lu_out.at[pl.ds(c0, w)], sem).wait()

    def process(c0, cb, lb, w):
        for g in range(_CW // _L):
            cbidx[pl.ds(g * _L, _L)] = zeros

        def cscan(g, ccnt):
            base = g * _L
            v = mid[pl.ds(base, _L)]
            valid = (base + lax.iota(jnp.int32, _L)) < cnt
            m = (v >= c0) & (v < c0 + w) & valid
            plsc.store_compressed(cmid.at[pl.ds(ccnt, _L)], v, mask=m)
            plsc.store_compressed(cbidx.at[pl.ds(ccnt, _L)],
                                  mbidx[pl.ds(base, _L)], mask=m)
            plsc.store_compressed(cf.at[pl.ds(ccnt, _L)],
                                  mf[pl.ds(base, _L)], mask=m)
            return ccnt + plsc.all_reduce_population_count(m)[0]

        ccnt = lax.fori_loop(0, (cnt + _L - 1) // _L, cscan, 0)
        ntb = (ccnt + 127) // 128

        def tsg(b, carry):
            pltpu.async_copy(ts_hbm.at[cbidx.at[pl.ds(b * 128, 128)]],
                             cts.at[pl.ds(b * 128, 128)], st)
            return carry

        lax.fori_loop(0, ntb, tsg, 0)

        def msg_fire(j, carry):
            b = _dscalar(cbidx, j)
            pltpu.async_copy(msg_hbm.at[pl.ds(b * _D, _D)],
                             msgbuf.at[j], sm)
            return carry

        lax.fori_loop(0, ccnt, msg_fire, 0)

        def msg_drain(j, carry):
            pltpu.make_async_copy(msg_hbm.at[pl.ds(0, _D)],
                                  msgbuf.at[0], sm).wait()
            return carry

        lax.fori_loop(0, ccnt, msg_drain, 0)

        def apply(j, carry):
            cl = _dsplat(cmid, j) - c0
            fs = _dsplat(cf, j)
            for q in range(_D // _L):
                rows = lax.iota(jnp.int32, _L) + q * _L
                old = plsc.load_gather(cb, [rows, cl])
                msgv = msgbuf[j, pl.ds(q * _L, _L)]
                plsc.store_scatter(cb, [rows, cl], msgv + fs * old)
            return carry

        lax.fori_loop(0, ccnt, apply, 0)

        def tsg_drain(b, carry):
            pltpu.make_async_copy(ts_hbm.at[pl.ds(0, 128)],
                                  cts.at[pl.ds(0, 128)], st).wait()
            return carry

        lax.fori_loop(0, ntb, tsg_drain, 0)

        def luapp(g, carry):
            base = g * _L
            v = cmid[pl.ds(base, _L)] - c0
            valid = (base + lax.iota(jnp.int32, _L)) < ccnt
            plsc.store_scatter(lb, [v], cts[pl.ds(base, _L)], mask=valid)
            return carry

        lax.fori_loop(0, (ccnt + _L - 1) // _L, luapp, 0)
        return ccnt

    nch = (_TILE * _TPW) // _W
    npair = nch // 2

    fire_in(lo, colbuf0, lubuf0, si0, _W)

    def pair(pr, carry):
        c0 = lo + (2 * pr) * _W
        c1 = c0 + _W

        @pl.when(pr > 0)
        def _w1():
            wait_out(c1 - 2 * _W, colbuf1, lubuf1, so1, _W)

        fire_in(c1, colbuf1, lubuf1, si1, _W)
        wait_in(c0, colbuf0, lubuf0, si0, _W)
        process(c0, colbuf0, lubuf0, _W)
        fire_out(c0, colbuf0, lubuf0, so0, _W)
        wait_in(c1, colbuf1, lubuf1, si1, _W)
        process(c1, colbuf1, lubuf1, _W)
        fire_out(c1, colbuf1, lubuf1, so1, _W)
        wait_out(c0, colbuf0, lubuf0, so0, _W)

        @pl.when(pr < npair - 1)
        def _f0():
            fire_in(c0 + 2 * _W, colbuf0, lubuf0, si0, _W)

        return carry

    lax.fori_loop(0, npair, pair, 0)
    wait_out(lo + (nch - 1) * _W, colbuf1, lubuf1, so1, _W)

    @pl.when(wid < 4)
    def _rem():
        c0 = lo + nch * _W
        fire_in(c0, colbuf0, lubuf0, si0, _TILE)
        wait_in(c0, colbuf0, lubuf0, si0, _TILE)
        process(c0, colbuf0, lubuf0, _TILE)
        fire_out(c0, colbuf0, lubuf0, so0, _TILE)
        wait_out(c0, colbuf0, lubuf0, so0, _TILE)

    @pl.when(wid == _NW - 1)
    def _lu_tail():
        c0 = lo + nch * _W
        w = _M - _TILE * _NFULL
        pltpu.async_copy(lu_in.at[pl.ds(c0, w)],
                         lubuf0.at[pl.ds(0, w)], si0).wait()

        def cscan(g, ccnt):
            base = g * _L
            v = mid[pl.ds(base, _L)]
            valid = (base + lax.iota(jnp.int32, _L)) < cnt
            m = (v >= c0) & valid
            plsc.store_compressed(cmid.at[pl.ds(ccnt, _L)], v, mask=m)
            plsc.store_compressed(cbidx.at[pl.ds(ccnt, _L)],
                                  mbidx[pl.ds(base, _L)], mask=m)
            return ccnt + plsc.all_reduce_population_count(m)[0]

        for g in range(_CW // _L):
            cbidx[pl.ds(g * _L, _L)] = zeros
        ccnt = lax.fori_loop(0, (cnt + _L - 1) // _L, cscan, 0)
        pltpu.async_copy(ts_hbm.at[cbidx.at[pl.ds(0, 128)]],
                         cts.at[pl.ds(0, 128)], st).wait()

        def luapp(g, carry):
            base = g * _L
            v = cmid[pl.ds(base, _L)] - c0
            valid = (base + lax.iota(jnp.int32, _L)) < ccnt
            plsc.store_scatter(lubuf0, [v], cts[pl.ds(base, _L)],
                               mask=valid)
            return carry

        lax.fori_loop(0, (ccnt + _L - 1) // _L, luapp, 0)
        pltpu.async_copy(lubuf0.at[pl.ds(0, w)],
                         lu_out.at[pl.ds(c0, w)], so0).wait()


_TC0 = _TILE * _NFULL
_TW = _M - _TC0
_TWB = 128
_KB = 1024
_KG = _B // _KB


def _tc_tail_body(memT_ref, msgT_ref, ids_ref, ts_ref, lut_ref, alias_ref,
                  out_ref, acc_msg, acc_ts, acc_hit):
    k = pl.program_id(0)

    @pl.when(k == 0)
    def _init():
        acc_msg[...] = jnp.zeros((_D, _TWB), jnp.float32)
        acc_ts[...] = jnp.zeros((1, _TWB), jnp.float32)
        acc_hit[...] = jnp.zeros((1, _TWB), jnp.float32)

    ids_b = ids_ref[0]
    ts_b = ts_ref[0]
    tgt = _TC0 + lax.broadcasted_iota(jnp.int32, (_TWB, 1), 0)
    oh = (ids_b == tgt).astype(jnp.float32)
    dn = (((1,), (1,)), ((), ()))
    acc_msg[...] += lax.dot_general(msgT_ref[...], oh, dn,
                                    preferred_element_type=jnp.float32)
    acc_ts[...] += lax.dot_general(ts_b, oh, dn,
                                   preferred_element_type=jnp.float32)
    ones = jnp.ones((1, _KB), jnp.float32)
    acc_hit[...] += lax.dot_general(ones, oh, dn,
                                    preferred_element_type=jnp.float32)

    @pl.when(k == _KG - 1)
    def _finish():
        old = memT_ref[...]
        lut = lut_ref[...]
        f = jnp.exp((lut - acc_ts[...]) * (1.0 / _LAMB))
        new = acc_msg[...] + f * old
        out_ref[...] = jnp.where(acc_hit[...] > 0.0, new, old)


_tc_tail = pl.pallas_call(
    _tc_tail_body,
    grid=(_KG,),
    in_specs=[
        pl.BlockSpec((_D, _TWB), lambda k: (0, _TC0 // _TWB)),
        pl.BlockSpec((_D, _KB), lambda k: (0, k)),
        pl.BlockSpec((1, 1, _KB), lambda k: (k, 0, 0)),
        pl.BlockSpec((1, 1, _KB), lambda k: (k, 0, 0)),
        pl.BlockSpec((1, _TWB), lambda k: (0, 0)),
        pl.BlockSpec((_D, _TWB), lambda k: (0, _TC0 // _TWB)),
    ],
    out_specs=pl.BlockSpec((_D, _TWB), lambda k: (0, _TC0 // _TWB)),
    out_shape=jax.ShapeDtypeStruct((_D, _M), jnp.float32),
    scratch_shapes=[
        pltpu.VMEM((_D, _TWB), jnp.float32),
        pltpu.VMEM((1, _TWB), jnp.float32),
        pltpu.VMEM((1, _TWB), jnp.float32),
    ],
    input_output_aliases={5: 0},
)


def kernel(memory, last_update, unique_node_ids, unique_messages, timestamps):
    memT = memory.T
    msg_flat = unique_messages.reshape(-1)
    outT, lu_out = _sc_update(memT, last_update, unique_node_ids,
                              msg_flat, timestamps)
    ids3 = unique_node_ids.reshape(_KG, 1, _KB)
    ts3 = timestamps.reshape(_KG, 1, _KB)
    lut = jnp.pad(lax.dynamic_slice(last_update, (_TC0,), (_TW,)),
                  (0, _TWB - _TW)).reshape(1, _TWB)
    outT = _tc_tail(memT, unique_messages.T, ids3, ts3, lut, outT)
    return outT.T, lu_out


# speedup vs baseline: 2.0293x; 2.0134x over previous
import functools

import jax
import jax.numpy as jnp
from jax import lax
from jax.experimental import pallas as pl
from jax.experimental.pallas import tpu as pltpu
from jax.experimental.pallas import tpu_sc as plsc

_M = 1000000
_D = 64
_B = 16384
_LAMB = 10.0
_L = 16
_NC = 2
_NS = 16
_NW = _NC * _NS
_CHUNK = 128
_CPW = _B // (_NW * _CHUNK)

_mesh = plsc.VectorSubcoreMesh(core_axis_name="c", subcore_axis_name="s")

_SPLAT_DNUMS = lax.GatherDimensionNumbers(
    offset_dims=(), collapsed_slice_dims=(0,), start_index_map=(0,))


def _splat(vec, lane):
    idx = jnp.full((_L, 1), lane, jnp.int32)
    return lax.gather(vec, idx, _SPLAT_DNUMS, (1,),
                      mode=lax.GatherScatterMode.PROMISE_IN_BOUNDS)


@functools.partial(
    pl.kernel,
    out_type=(),
    mesh=_mesh,
    compiler_params=pltpu.CompilerParams(use_tc_tiling_on_sc=False),
    scratch_types=[
        pltpu.VMEM((_CPW, _CHUNK), jnp.int32),
        pltpu.VMEM((_CPW, _CHUNK, _D), jnp.float32),
        pltpu.VMEM((_CPW, _CHUNK, _D), jnp.float32),
        pltpu.VMEM((_CPW, _CHUNK), jnp.float32),
        pltpu.VMEM((_CPW, _CHUNK), jnp.float32),
        pltpu.SemaphoreType.DMA,
        pltpu.SemaphoreType.DMA,
    ],
)
def _sc_update(mem_ref, lu_ref, ids_hbm, msg_hbm, ts_hbm,
               idx_v, msg_v, rows_v, ts_v, lu_v, sem_rows, sem_sc):
    wid = lax.axis_index("s") * _NC + lax.axis_index("c")
    cbase = wid * _CPW
    pltpu.sync_copy(ids_hbm.at[pl.ds(cbase, _CPW)], idx_v)
    pltpu.sync_copy(msg_hbm.at[pl.ds(cbase, _CPW)], msg_v)
    pltpu.sync_copy(ts_hbm.at[pl.ds(cbase, _CPW)], ts_v)
    copies = []
    for j in range(_CPW):
        copies.append(
            pltpu.async_copy(mem_ref.at[idx_v.at[j]], rows_v.at[j], sem_rows))
        copies.append(
            pltpu.async_copy(lu_ref.at[idx_v.at[j]], lu_v.at[j], sem_sc))
    for c in copies:
        c.wait()

    for j in range(_CPW):
        def grp_body(g, carry, j=j):
            r0 = g * _L
            f = jnp.exp((lu_v[j, pl.ds(r0, _L)] - ts_v[j, pl.ds(r0, _L)])
                        * (1.0 / _LAMB))
            for r in range(_L):
                spl = _splat(f, r)
                row = r0 + r
                for c0 in range(0, _D, _L):
                    sl = pl.ds(c0, _L)
                    rows_v[j, row, sl] = (msg_v[j, row, sl]
                                          + spl * rows_v[j, row, sl])
            return carry
        lax.fori_loop(0, _CHUNK // _L, grp_body, 0)

    copies = []
    for j in range(_CPW):
        copies.append(
            pltpu.async_copy(rows_v.at[j], mem_ref.at[idx_v.at[j]], sem_rows))
        copies.append(
            pltpu.async_copy(ts_v.at[j], lu_ref.at[idx_v.at[j]], sem_sc))
    for c in copies:
        c.wait()


def kernel(memory, last_update, unique_node_ids, unique_messages, timestamps):
    ids2 = unique_node_ids.reshape(_NW * _CPW, _CHUNK)
    msg3 = unique_messages.reshape(_NW * _CPW, _CHUNK, _D)
    ts2 = timestamps.reshape(_NW * _CPW, _CHUNK)
    mem_ref = jax.new_ref(memory)
    lu_ref = jax.new_ref(last_update)
    _sc_update(mem_ref, lu_ref, ids2, msg3, ts2)
    return jax.freeze(mem_ref), jax.freeze(lu_ref)
